# baseline (speedup 1.0000x reference)
import functools

import jax
import jax.numpy as jnp
from jax import lax
from jax.experimental import pallas as pl
from jax.experimental.pallas import tpu as pltpu
from jax.experimental.pallas import tpu_sc as plsc

NUM_Q = 8
K = 1024
D = 256
BM = 1152
HM = BM // 2


def _half_scores(r, xx, cb_ref, norms_s):
    mm = jax.lax.dot_general(
        r.astype(jnp.bfloat16), cb_ref[...].astype(jnp.bfloat16),
        (((1,), (1,)), ((), ())),
        preferred_element_type=jnp.float32)
    scores = -(xx - 2.0 * mm + norms_s[...])
    m = jnp.max(scores, axis=1, keepdims=True)
    iota_k = jax.lax.broadcasted_iota(jnp.int32, (HM, K), 1)
    return jnp.min(jnp.where(scores == m, iota_k, K), axis=1)


def _score_kernel(r_ref, q_ref, xx_ref, cb_ref, idx_ref, rout_ref, norms_s):
    @pl.when(pl.program_id(0) == 0)
    def _():
        embed = cb_ref[...]
        norms_s[...] = jnp.sum(embed * embed, axis=1)[None, :]

    r = r_ref[...] - q_ref[...]
    idx_a = _half_scores(r[:HM, :], xx_ref[:HM, :], cb_ref, norms_s)
    idx_b = _half_scores(r[HM:, :], xx_ref[HM:, :], cb_ref, norms_s)
    idx_ref[:HM, :] = idx_a[:, None]
    idx_ref[HM:, :] = idx_b[:, None]
    rout_ref[...] = r


def _quant_kernel(x_ref, r_ref, q_ref, out_ref):
    out_ref[...] = x_ref[...] - (r_ref[...] - q_ref[...])


def _make_score(n):
    return pl.pallas_call(
        _score_kernel,
        grid=(n // BM,),
        in_specs=[
            pl.BlockSpec((BM, D), lambda b: (b, 0)),
            pl.BlockSpec((BM, D), lambda b: (b, 0)),
            pl.BlockSpec((BM, 1), lambda b: (b, 0)),
            pl.BlockSpec((K, D), lambda b: (0, 0)),
        ],
        out_specs=[
            pl.BlockSpec((BM, 1), lambda b: (b, 0)),
            pl.BlockSpec((BM, D), lambda b: (b, 0)),
        ],
        out_shape=[
            jax.ShapeDtypeStruct((n, 1), jnp.int32),
            jax.ShapeDtypeStruct((n, D), jnp.float32),
        ],
        scratch_shapes=[pltpu.VMEM((1, K), jnp.float32)],
    )


def _make_quant(n):
    spec = pl.BlockSpec((BM, D), lambda b: (b, 0))
    return pl.pallas_call(
        _quant_kernel,
        grid=(n // BM,),
        in_specs=[spec, spec, spec],
        out_specs=spec,
        out_shape=jax.ShapeDtypeStruct((n, D), jnp.float32),
    )


def _make_gather(n):
    info = plsc.get_sparse_core_info()
    nw = info.num_cores * info.num_subcores
    b_per_w = n // nw
    mesh = plsc.VectorSubcoreMesh(core_axis_name="c", subcore_axis_name="s")

    @functools.partial(
        pl.kernel, mesh=mesh,
        out_type=jax.ShapeDtypeStruct((n, D), jnp.float32),
        scratch_types=[
            pltpu.VMEM((b_per_w,), jnp.int32),
            pltpu.VMEM((b_per_w, D), jnp.float32),
            pltpu.SemaphoreType.DMA,
        ],
    )
    def gather(table_hbm, idx_hbm, out_hbm, idx_v, rows_v, sem):
        wid = lax.axis_index("s") * info.num_cores + lax.axis_index("c")
        base = wid * b_per_w
        pltpu.sync_copy(idx_hbm.at[pl.ds(base, b_per_w)], idx_v)
        pltpu.async_copy(table_hbm.at[idx_v], rows_v, sem).wait()
        pltpu.sync_copy(rows_v, out_hbm.at[pl.ds(base, b_per_w)])

    return gather


@jax.jit
def kernel(x, codebooks):
    shape = x.shape
    n = shape[0] * shape[1]
    half = n // 2
    x_flat = x.reshape(n, D)
    score_stage = _make_score(half)
    gather_sc = _make_gather(half)
    quant_stage = _make_quant(n)
    xs = (x_flat[:half], x_flat[half:])
    rs = list(xs)
    qs = [jnp.zeros_like(xs[0]), jnp.zeros_like(xs[1])]
    idxs = [[], []]
    for i in range(NUM_Q):
        for h in (0, 1):
            if i == 0:
                xx = jnp.sum(xs[h] * xs[h], axis=1, keepdims=True)
            else:
                rq = rs[h] - qs[h]
                xx = jnp.sum(rq * rq, axis=1, keepdims=True)
            idx_i, rs[h] = score_stage(rs[h], qs[h], xx, codebooks[i])
            qs[h] = gather_sc(codebooks[i], idx_i[:, 0])
            idxs[h].append(idx_i[:, 0])
    quant = quant_stage(x_flat, jnp.concatenate(rs), jnp.concatenate(qs))
    indices = jnp.concatenate(
        [jnp.stack(idxs[0]), jnp.stack(idxs[1])], axis=1)
    return (indices.reshape(NUM_Q, shape[0], shape[1]),
            quant.reshape(shape))


# speedup vs baseline: 1.0802x; 1.0802x over previous
import functools

import jax
import jax.numpy as jnp
from jax import lax
from jax.experimental import pallas as pl
from jax.experimental.pallas import tpu as pltpu
from jax.experimental.pallas import tpu_sc as plsc

NUM_Q = 8
K = 1024
D = 256
BM = 1152
HM = BM // 2


def _half_scores(r, xx, cb_ref, norms_s):
    mm = jax.lax.dot_general(
        r.astype(jnp.bfloat16), cb_ref[...].astype(jnp.bfloat16),
        (((1,), (1,)), ((), ())),
        preferred_element_type=jnp.float32)
    scores = -(xx - 2.0 * mm + norms_s[...])
    m = jnp.max(scores, axis=1, keepdims=True)
    iota_k = jax.lax.broadcasted_iota(jnp.int32, (HM, K), 1)
    return jnp.min(jnp.where(scores == m, iota_k, K), axis=1)


def _score_kernel(r_ref, q_ref, xx_ref, cb_ref, idx_ref, rout_ref, norms_s):
    @pl.when(pl.program_id(0) == 0)
    def _():
        embed = cb_ref[...]
        norms_s[...] = jnp.sum(embed * embed, axis=1)[None, :]

    r = r_ref[...] - q_ref[...]
    idx_a = _half_scores(r[:HM, :], xx_ref[:HM, :], cb_ref, norms_s)
    idx_b = _half_scores(r[HM:, :], xx_ref[HM:, :], cb_ref, norms_s)
    idx_ref[:HM, :] = idx_a[:, None]
    idx_ref[HM:, :] = idx_b[:, None]
    rout_ref[...] = r


def _quant_kernel(x_ref, r_ref, q_ref, out_ref):
    out_ref[...] = x_ref[...] - (r_ref[...] - q_ref[...])


def _make_score(n):
    return pl.pallas_call(
        _score_kernel,
        grid=(n // BM,),
        in_specs=[
            pl.BlockSpec((BM, D), lambda b: (b, 0)),
            pl.BlockSpec((BM, D), lambda b: (b, 0)),
            pl.BlockSpec((BM, 1), lambda b: (b, 0)),
            pl.BlockSpec((K, D), lambda b: (0, 0)),
        ],
        out_specs=[
            pl.BlockSpec((BM, 1), lambda b: (b, 0)),
            pl.BlockSpec((BM, D), lambda b: (b, 0)),
        ],
        out_shape=[
            jax.ShapeDtypeStruct((n, 1), jnp.int32),
            jax.ShapeDtypeStruct((n, D), jnp.float32),
        ],
        scratch_shapes=[pltpu.VMEM((1, K), jnp.float32)],
    )


def _make_quant(n):
    spec = pl.BlockSpec((BM, D), lambda b: (b, 0))
    return pl.pallas_call(
        _quant_kernel,
        grid=(n // BM,),
        in_specs=[spec, spec, spec],
        out_specs=spec,
        out_shape=jax.ShapeDtypeStruct((n, D), jnp.float32),
    )


def _make_gather(n):
    info = plsc.get_sparse_core_info()
    nw = info.num_cores * info.num_subcores
    b_per_w = n // nw
    mesh = plsc.VectorSubcoreMesh(core_axis_name="c", subcore_axis_name="s")

    @functools.partial(
        pl.kernel, mesh=mesh,
        out_type=jax.ShapeDtypeStruct((n, D), jnp.float32),
        scratch_types=[
            pltpu.VMEM((b_per_w,), jnp.int32),
            pltpu.VMEM((b_per_w, D), jnp.float32),
            pltpu.SemaphoreType.DMA,
        ],
    )
    def gather(table_hbm, idx_hbm, out_hbm, idx_v, rows_v, sem):
        wid = lax.axis_index("s") * info.num_cores + lax.axis_index("c")
        base = wid * b_per_w
        pltpu.sync_copy(idx_hbm.at[pl.ds(base, b_per_w)], idx_v)
        pltpu.async_copy(table_hbm.at[idx_v], rows_v, sem).wait()
        pltpu.sync_copy(rows_v, out_hbm.at[pl.ds(base, b_per_w)])

    return gather


@jax.jit
def kernel(x, codebooks):
    shape = x.shape
    n = shape[0] * shape[1]
    x_flat = x.reshape(n, D)
    score_stage = _make_score(n)
    gather_sc = _make_gather(n)
    quant_stage = _make_quant(n)
    r = x_flat
    q = jnp.zeros_like(x_flat)
    idxs = []
    for i in range(NUM_Q):
        if i == 0:
            xx = jnp.sum(x_flat * x_flat, axis=1, keepdims=True)
        else:
            xx = jnp.sum((r - q) * (r - q), axis=1, keepdims=True)
        idx_i, r = score_stage(r, q, xx, codebooks[i])
        q = gather_sc(codebooks[i], idx_i[:, 0])
        idxs.append(idx_i[:, 0])
    quant = quant_stage(x_flat, r, q)
    indices = jnp.stack(idxs).reshape(NUM_Q, shape[0], shape[1])
    return indices, quant.reshape(shape)
